# baseline (speedup 1.0000x reference)
import jax
import jax.numpy as jnp
from jax import lax
from jax.experimental import pallas as pl
from jax.experimental.pallas import tpu as pltpu

_N_NET = 10
_RBLK = 200


def _pool_kernel(wf_ref, gf_ref, w_ref, g_ref, x_ref, o_ref, m_ref, s_ref):
    @pl.when(pl.program_id(0) == 0)
    def _():
        w = wf_ref[:, :]
        g = gf_ref[:, :]
        n_roi = w.shape[1]
        row = lax.broadcasted_iota(jnp.int32, (_N_NET, n_roi), 0)
        mask = g == row
        s_masked = jnp.where(mask, w, -jnp.inf)
        m = jnp.max(s_masked, axis=1, keepdims=True)
        m = jnp.where(jnp.isfinite(m), m, 0.0)
        e = jnp.where(mask, jnp.exp(w - m), 0.0)
        s = jnp.sum(e, axis=1, keepdims=True)
        m_ref[:, :] = m
        s_ref[:, :] = jnp.where(s == 0.0, 1.0, s)

    wb = w_ref[0, :, :]
    gb = g_ref[0, :, :]
    rowb = lax.broadcasted_iota(jnp.int32, (_N_NET, _RBLK), 0)
    b = jnp.where(gb == rowb, jnp.exp(wb - m_ref[:, :]), 0.0) / s_ref[:, :]
    part = jnp.dot(b, x_ref[:, :], preferred_element_type=jnp.float32)

    @pl.when(pl.program_id(0) == 0)
    def _():
        o_ref[:, :] = part

    @pl.when(pl.program_id(0) > 0)
    def _():
        o_ref[:, :] += part


def kernel(x, raw_weights, group):
    n_roi, feat = x.shape
    w2 = raw_weights.reshape(1, n_roi)
    g2 = group.reshape(1, n_roi).astype(jnp.int32)
    grid = n_roi // _RBLK
    return pl.pallas_call(
        _pool_kernel,
        grid=(grid,),
        in_specs=[
            pl.BlockSpec((1, n_roi), lambda i: (0, 0)),
            pl.BlockSpec((1, n_roi), lambda i: (0, 0)),
            pl.BlockSpec((1, 1, _RBLK), lambda i: (i, 0, 0)),
            pl.BlockSpec((1, 1, _RBLK), lambda i: (i, 0, 0)),
            pl.BlockSpec((_RBLK, feat), lambda i: (i, 0)),
        ],
        out_specs=pl.BlockSpec((_N_NET, feat), lambda i: (0, 0)),
        scratch_shapes=[
            pltpu.VMEM((_N_NET, 1), jnp.float32),
            pltpu.VMEM((_N_NET, 1), jnp.float32),
        ],
        out_shape=jax.ShapeDtypeStruct((_N_NET, feat), jnp.float32),
    )(w2, g2, w2.reshape(grid, 1, _RBLK), g2.reshape(grid, 1, _RBLK), x)


# speedup vs baseline: 2.2689x; 2.2689x over previous
import jax
import jax.numpy as jnp
from jax import lax
from jax.experimental import pallas as pl
from jax.experimental.pallas import tpu as pltpu

_N_NET = 10
_N_CHUNK = 5


def _pool_kernel(w_ref, g_ref, x_hbm, o_ref, xv, sems):
    n_roi = xv.shape[0]
    rows = n_roi // _N_CHUNK
    copies = []
    for k in range(_N_CHUNK):
        c = pltpu.make_async_copy(
            x_hbm.at[pl.ds(k * rows, rows), :],
            xv.at[pl.ds(k * rows, rows), :],
            sems.at[k],
        )
        c.start()
        copies.append(c)

    w = w_ref[:, :]
    g = g_ref[:, :]
    row = lax.broadcasted_iota(jnp.int32, (_N_NET, n_roi), 0)
    mask = g == row
    s_masked = jnp.where(mask, w, -jnp.inf)
    m = jnp.max(s_masked, axis=1, keepdims=True)
    m = jnp.where(jnp.isfinite(m), m, 0.0)
    e = jnp.where(mask, jnp.exp(w - m), 0.0)
    s = jnp.sum(e, axis=1, keepdims=True)
    b = e / jnp.where(s == 0.0, 1.0, s)

    for c in copies:
        c.wait()
    o_ref[:, :] = jnp.dot(b, xv[:, :], preferred_element_type=jnp.float32)


def kernel(x, raw_weights, group):
    n_roi, feat = x.shape
    return pl.pallas_call(
        _pool_kernel,
        in_specs=[
            pl.BlockSpec((1, n_roi), lambda: (0, 0)),
            pl.BlockSpec((1, n_roi), lambda: (0, 0)),
            pl.BlockSpec(memory_space=pl.ANY),
        ],
        out_specs=pl.BlockSpec((_N_NET, feat), lambda: (0, 0)),
        scratch_shapes=[
            pltpu.VMEM((n_roi, feat), jnp.float32),
            pltpu.SemaphoreType.DMA((_N_CHUNK,)),
        ],
        out_shape=jax.ShapeDtypeStruct((_N_NET, feat), jnp.float32),
    )(raw_weights.reshape(1, n_roi), group.reshape(1, n_roi).astype(jnp.int32), x)
